# Initial kernel scaffold; baseline (speedup 1.0000x reference)
import jax
import jax.numpy as jnp
from jax.experimental import pallas as pl


def kernel(x, edge_index, W1, b1, W2, b2):
    raise NotImplementedError("write your pallas kernel here")


# speedup vs baseline: 7.5613x; 7.5613x over previous
import functools

import jax
import jax.numpy as jnp
from jax import lax
from jax.experimental import pallas as pl
from jax.experimental.pallas import tpu as pltpu
from jax.experimental.pallas import tpu_sc as plsc

ALPHA = 0.1
K = 10
L = 16
NC, NS = 2, 16
NW = NC * NS


def _fast_rsqrt(d):
    i = lax.bitcast_convert_type(d, jnp.int32)
    i = jnp.int32(0x5F3759DF) - lax.shift_right_arithmetic(i, 1)
    y = lax.bitcast_convert_type(i, jnp.float32)
    for _ in range(3):
        y = y * (1.5 - 0.5 * d * y * y)
    return y


def _make_prop(n, e, f, n_chunks):
    fp = f // NW
    ch = e // n_chunks
    assert fp * NW == f and ch * n_chunks == e and ch % L == 0 and ch % 8 == 0
    n_grp = n // L
    c_grp = ch // L
    mesh = plsc.VectorSubcoreMesh(
        core_axis_name="c", subcore_axis_name="s", num_cores=NC, num_subcores=NS
    )

    @functools.partial(
        pl.kernel,
        mesh=mesh,
        compiler_params=pltpu.CompilerParams(needs_layout_passes=False),
        out_type=jax.ShapeDtypeStruct((f, n), jnp.float32),
        scratch_types=(
            [pltpu.VMEM((n,), jnp.float32)]
            + [pltpu.VMEM((n,), jnp.float32)] * fp
            + [pltpu.VMEM((n,), jnp.float32)] * fp
            + [pltpu.VMEM((n,), jnp.float32)] * fp
            + [pltpu.VMEM((ch,), jnp.int32)] * 4
            + [
                pltpu.SemaphoreType.DMA,
                pltpu.SemaphoreType.DMA,
            ]
        ),
    )
    def prop(h0t_hbm, src_hbm, dst_hbm, out_hbm, *rest):
        dinv_v = rest[0]
        h0_v = rest[1:1 + fp]
        zt_v = rest[1 + fp:1 + 2 * fp]
        acc_v = rest[1 + 2 * fp:1 + 3 * fp]
        sb0, sb1, db0, db1, sem0, sem1 = rest[1 + 3 * fp:]
        sb = (sb0, sb1)
        db = (db0, db1)
        wid = lax.axis_index("s") * NC + lax.axis_index("c")
        f0 = wid * fp
        sems = (sem0, sem1)

        def start_chunk(c, p):
            a = pltpu.async_copy(src_hbm.at[pl.ds(c * ch, ch)], sb[p], sems[p])
            b = pltpu.async_copy(dst_hbm.at[pl.ds(c * ch, ch)], db[p], sems[p])
            return (a, b)

        def edge_pass(proc):
            pend = start_chunk(0, 0)
            for c in range(n_chunks):
                p = c & 1
                cur = pend
                if c + 1 < n_chunks:
                    pend = start_chunk(c + 1, p ^ 1)
                cur[0].wait()
                cur[1].wait()

                def grp_body(g, _, _p=p):
                    sv = sb[_p][pl.ds(g * L, L)]
                    dv = db[_p][pl.ds(g * L, L)]
                    proc(sv, dv)
                    return 0

                lax.fori_loop(0, c_grp, grp_body, 0, unroll=4)

        ones = jnp.full((L,), 1.0, jnp.float32)

        def init_deg(g, _):
            acc_v[0][pl.ds(g * L, L)] = ones
            return 0

        lax.fori_loop(0, n_grp, init_deg, 0, unroll=4)

        def hist(sv, dv):
            plsc.addupdate_scatter(acc_v[0], [dv], ones)

        edge_pass(hist)

        def calc_dinv(g, _):
            s = pl.ds(g * L, L)
            dinv_v[s] = _fast_rsqrt(acc_v[0][s])
            return 0

        lax.fori_loop(0, n_grp, calc_dinv, 0, unroll=2)

        for j in range(fp):
            pltpu.sync_copy(h0t_hbm.at[f0 + j], h0_v[j])

        def init_zt(g, _):
            s = pl.ds(g * L, L)
            dv = dinv_v[s]
            for j in range(fp):
                zt_v[j][s] = dv * h0_v[j][s]
            return 0

        lax.fori_loop(0, n_grp, init_zt, 0, unroll=2)

        def zero_acc(g, _):
            s = pl.ds(g * L, L)
            for j in range(fp):
                acc_v[j][s] = jnp.zeros((L,), jnp.float32)
            return 0

        def scatter_edges(sv, dv):
            for j in range(fp):
                vals = plsc.load_gather(zt_v[j], [sv])
                plsc.addupdate_scatter(acc_v[j], [dv], vals)

        def one_iter(last):
            lax.fori_loop(0, n_grp, zero_acc, 0, unroll=4)
            edge_pass(scatter_edges)

            def upd(g, _):
                s = pl.ds(g * L, L)
                dv = dinv_v[s]
                for j in range(fp):
                    z = ((1.0 - ALPHA) * dv * (acc_v[j][s] + zt_v[j][s])
                         + ALPHA * h0_v[j][s])
                    zt_v[j][s] = z if last else dv * z
                return 0

            lax.fori_loop(0, n_grp, upd, 0, unroll=2)

        def k_body(k, c):
            one_iter(False)
            return c

        lax.fori_loop(0, K - 1, k_body, 0)
        one_iter(True)

        for j in range(fp):
            pltpu.sync_copy(zt_v[j], out_hbm.at[f0 + j])

    return prop


def _make_mm(m, kdim, ndim, relu_in, blk_m):
    assert m % blk_m == 0

    def body(x_ref, w_ref, b_ref, o_ref):
        xv = x_ref[...]
        if relu_in:
            xv = jnp.maximum(xv, 0.0)
        o_ref[...] = (
            jnp.dot(xv, w_ref[...], preferred_element_type=jnp.float32)
            + b_ref[...]
        )

    return pl.pallas_call(
        body,
        grid=(m // blk_m,),
        in_specs=[
            pl.BlockSpec((blk_m, kdim), lambda i: (i, 0)),
            pl.BlockSpec((kdim, ndim), lambda i: (0, 0)),
            pl.BlockSpec((1, ndim), lambda i: (0, 0)),
        ],
        out_specs=pl.BlockSpec((blk_m, ndim), lambda i: (i, 0)),
        out_shape=jax.ShapeDtypeStruct((m, ndim), jnp.float32),
    )


def kernel(x, edge_index, W1, b1, W2, b2):
    n, d_in = x.shape
    e = edge_index.shape[1]
    hid = W1.shape[1]
    d_out = W2.shape[1]

    src = edge_index[0]
    dst = edge_index[1]

    mm1 = _make_mm(n, d_in, hid, relu_in=False, blk_m=1000)
    mm2 = _make_mm(n, hid, d_out, relu_in=True, blk_m=1000)
    prop1 = _make_prop(n, e, hid, n_chunks=20)
    prop2 = _make_prop(n, e, d_out, n_chunks=20)

    h0 = mm1(x, W1, b1.reshape(1, hid))
    z1t = prop1(h0.T, src, dst)
    h2 = mm2(z1t.T, W2, b2.reshape(1, d_out))
    outt = prop2(h2.T, src, dst)
    return outt.T


# speedup vs baseline: 20.0175x; 2.6473x over previous
import functools

import jax
import jax.numpy as jnp
from jax import lax
from jax.experimental import pallas as pl
from jax.experimental.pallas import tpu as pltpu
from jax.experimental.pallas import tpu_sc as plsc

ALPHA = 0.1
K = 10
L = 16
NC, NS = 2, 16
NW = NC * NS


def _fast_rsqrt(d):
    i = lax.bitcast_convert_type(d, jnp.int32)
    i = jnp.int32(0x5F3759DF) - lax.shift_right_arithmetic(i, 1)
    y = lax.bitcast_convert_type(i, jnp.float32)
    for _ in range(3):
        y = y * (1.5 - 0.5 * d * y * y)
    return y


def _make_prop(n, e, f, n_chunks):
    fp = f // NW
    ch = e // n_chunks
    assert fp * NW == f and ch * n_chunks == e and ch % L == 0 and ch % 8 == 0
    n_grp = n // L
    c_grp = ch // L
    mesh = plsc.VectorSubcoreMesh(
        core_axis_name="c", subcore_axis_name="s", num_cores=NC, num_subcores=NS
    )

    @functools.partial(
        pl.kernel,
        mesh=mesh,
        compiler_params=pltpu.CompilerParams(needs_layout_passes=False),
        out_type=jax.ShapeDtypeStruct((f, n), jnp.float32),
        scratch_types=(
            [pltpu.VMEM((n,), jnp.float32)]
            + [pltpu.VMEM((n,), jnp.float32)] * fp
            + [pltpu.VMEM((n,), jnp.float32)] * fp
            + [pltpu.VMEM((n,), jnp.float32)] * fp
            + [pltpu.VMEM((ch,), jnp.int32)] * 4
            + [
                pltpu.SemaphoreType.DMA,
                pltpu.SemaphoreType.DMA,
            ]
        ),
    )
    def prop(h0t_hbm, src_hbm, dst_hbm, out_hbm, *rest):
        dinv_v = rest[0]
        h0_v = rest[1:1 + fp]
        zt_v = rest[1 + fp:1 + 2 * fp]
        acc_v = rest[1 + 2 * fp:1 + 3 * fp]
        sb0, sb1, db0, db1, sem0, sem1 = rest[1 + 3 * fp:]
        sb = (sb0, sb1)
        db = (db0, db1)
        wid = lax.axis_index("s") * NC + lax.axis_index("c")
        f0 = wid * fp
        sems = (sem0, sem1)

        def start_chunk(c, p):
            a = pltpu.async_copy(src_hbm.at[pl.ds(c * ch, ch)], sb[p], sems[p])
            b = pltpu.async_copy(dst_hbm.at[pl.ds(c * ch, ch)], db[p], sems[p])
            return (a, b)

        def edge_pass(proc):
            pend = start_chunk(0, 0)
            for c in range(n_chunks):
                p = c & 1
                cur = pend
                if c + 1 < n_chunks:
                    pend = start_chunk(c + 1, p ^ 1)
                cur[0].wait()
                cur[1].wait()

                @plsc.parallel_loop(0, ch, step=L, unroll=4)
                def grp_body(g, _p=p):
                    sv = sb[_p][pl.ds(g, L)]
                    dv = db[_p][pl.ds(g, L)]
                    proc(sv, dv)

        ones = jnp.full((L,), 1.0, jnp.float32)

        @plsc.parallel_loop(0, n, step=L, unroll=4)
        def init_deg(g):
            acc_v[0][pl.ds(g, L)] = ones

        def hist(sv, dv):
            plsc.addupdate_scatter(acc_v[0], [dv], ones)

        edge_pass(hist)

        @plsc.parallel_loop(0, n, step=L, unroll=2)
        def calc_dinv(g):
            s = pl.ds(g, L)
            dinv_v[s] = _fast_rsqrt(acc_v[0][s])

        for j in range(fp):
            pltpu.sync_copy(h0t_hbm.at[f0 + j], h0_v[j])

        @plsc.parallel_loop(0, n, step=L, unroll=2)
        def init_zt(g):
            s = pl.ds(g, L)
            dv = dinv_v[s]
            for j in range(fp):
                zt_v[j][s] = dv * h0_v[j][s]

        def scatter_edges(sv, dv):
            for j in range(fp):
                vals = plsc.load_gather(zt_v[j], [sv])
                plsc.addupdate_scatter(acc_v[j], [dv], vals)

        def one_iter(last):
            @plsc.parallel_loop(0, n, step=L, unroll=4)
            def zero_acc(g):
                s = pl.ds(g, L)
                for j in range(fp):
                    acc_v[j][s] = jnp.zeros((L,), jnp.float32)

            edge_pass(scatter_edges)

            @plsc.parallel_loop(0, n, step=L, unroll=2)
            def upd(g):
                s = pl.ds(g, L)
                dv = dinv_v[s]
                for j in range(fp):
                    z = ((1.0 - ALPHA) * dv * (acc_v[j][s] + zt_v[j][s])
                         + ALPHA * h0_v[j][s])
                    zt_v[j][s] = z if last else dv * z

        def k_body(k, c):
            one_iter(False)
            return c

        lax.fori_loop(0, K - 1, k_body, 0)
        one_iter(True)

        for j in range(fp):
            pltpu.sync_copy(zt_v[j], out_hbm.at[f0 + j])

    return prop


def _make_mm(m, kdim, ndim, relu_in, blk_m):
    assert m % blk_m == 0

    def body(x_ref, w_ref, b_ref, o_ref):
        xv = x_ref[...]
        if relu_in:
            xv = jnp.maximum(xv, 0.0)
        o_ref[...] = (
            jnp.dot(xv, w_ref[...], preferred_element_type=jnp.float32)
            + b_ref[...]
        )

    return pl.pallas_call(
        body,
        grid=(m // blk_m,),
        in_specs=[
            pl.BlockSpec((blk_m, kdim), lambda i: (i, 0)),
            pl.BlockSpec((kdim, ndim), lambda i: (0, 0)),
            pl.BlockSpec((1, ndim), lambda i: (0, 0)),
        ],
        out_specs=pl.BlockSpec((blk_m, ndim), lambda i: (i, 0)),
        out_shape=jax.ShapeDtypeStruct((m, ndim), jnp.float32),
    )


def kernel(x, edge_index, W1, b1, W2, b2):
    n, d_in = x.shape
    e = edge_index.shape[1]
    hid = W1.shape[1]
    d_out = W2.shape[1]

    src = edge_index[0]
    dst = edge_index[1]

    mm1 = _make_mm(n, d_in, hid, relu_in=False, blk_m=1000)
    mm2 = _make_mm(n, hid, d_out, relu_in=True, blk_m=1000)
    prop1 = _make_prop(n, e, hid, n_chunks=20)
    prop2 = _make_prop(n, e, d_out, n_chunks=20)

    h0 = mm1(x, W1, b1.reshape(1, hid))
    z1t = prop1(h0.T, src, dst)
    h2 = mm2(z1t.T, W2, b2.reshape(1, d_out))
    outt = prop2(h2.T, src, dst)
    return outt.T


# speedup vs baseline: 22.2510x; 1.1116x over previous
import functools

import jax
import jax.numpy as jnp
from jax import lax
from jax.experimental import pallas as pl
from jax.experimental.pallas import tpu as pltpu
from jax.experimental.pallas import tpu_sc as plsc

ALPHA = 0.1
K = 10
L = 16
NC, NS = 2, 16
NW = NC * NS


def _fast_rsqrt(d):
    i = lax.bitcast_convert_type(d, jnp.int32)
    i = jnp.int32(0x5F3759DF) - lax.shift_right_arithmetic(i, 1)
    y = lax.bitcast_convert_type(i, jnp.float32)
    for _ in range(3):
        y = y * (1.5 - 0.5 * d * y * y)
    return y


def _make_prop(n, e, f, n_chunks):
    fp = f // NW
    ch = e // n_chunks
    assert fp * NW == f and ch * n_chunks == e and ch % L == 0 and ch % 8 == 0
    assert n <= (1 << 14)
    n_grp = n // L
    c_grp = ch // L
    mesh = plsc.VectorSubcoreMesh(
        core_axis_name="c", subcore_axis_name="s", num_cores=NC, num_subcores=NS
    )

    @functools.partial(
        pl.kernel,
        mesh=mesh,
        compiler_params=pltpu.CompilerParams(needs_layout_passes=False),
        out_type=jax.ShapeDtypeStruct((f, n), jnp.float32),
        scratch_types=(
            [pltpu.VMEM((n,), jnp.float32)]
            + [pltpu.VMEM((n,), jnp.float32)] * fp
            + [pltpu.VMEM((n,), jnp.float32)] * fp
            + [pltpu.VMEM((n,), jnp.float32)] * fp
            + [pltpu.VMEM((ch,), jnp.int32)] * 2
            + [
                pltpu.SemaphoreType.DMA,
                pltpu.SemaphoreType.DMA,
            ]
        ),
    )
    def prop(h0t_hbm, edge_hbm, out_hbm, *rest):
        dinv_v = rest[0]
        h0_v = rest[1:1 + fp]
        zt_v = rest[1 + fp:1 + 2 * fp]
        acc_v = rest[1 + 2 * fp:1 + 3 * fp]
        eb0, eb1, sem0, sem1 = rest[1 + 3 * fp:]
        eb = (eb0, eb1)
        wid = lax.axis_index("s") * NC + lax.axis_index("c")
        f0 = wid * fp
        sems = (sem0, sem1)

        def start_chunk(c, p):
            return pltpu.async_copy(
                edge_hbm.at[pl.ds(c * ch, ch)], eb[p], sems[p])

        def edge_pass(proc):
            pend = start_chunk(0, 0)
            for c in range(n_chunks):
                p = c & 1
                cur = pend
                if c + 1 < n_chunks:
                    pend = start_chunk(c + 1, p ^ 1)
                cur.wait()

                @plsc.parallel_loop(0, ch, step=L, unroll=8)
                def grp_body(g, _p=p):
                    pv = eb[_p][pl.ds(g, L)]
                    sv = lax.bitwise_and(pv, jnp.int32((1 << 14) - 1))
                    dv = lax.shift_right_logical(pv, jnp.int32(14))
                    proc(sv, dv)

        ones = jnp.full((L,), 1.0, jnp.float32)

        @plsc.parallel_loop(0, n, step=L, unroll=4)
        def init_deg(g):
            acc_v[0][pl.ds(g, L)] = ones

        def hist(sv, dv):
            plsc.addupdate_scatter(acc_v[0], [dv], ones)

        edge_pass(hist)

        @plsc.parallel_loop(0, n, step=L, unroll=2)
        def calc_dinv(g):
            s = pl.ds(g, L)
            dinv_v[s] = _fast_rsqrt(acc_v[0][s])

        for j in range(fp):
            pltpu.sync_copy(h0t_hbm.at[f0 + j], h0_v[j])

        @plsc.parallel_loop(0, n, step=L, unroll=2)
        def init_zt(g):
            s = pl.ds(g, L)
            dv = dinv_v[s]
            for j in range(fp):
                zt_v[j][s] = dv * h0_v[j][s]

        def scatter_edges(sv, dv):
            for j in range(fp):
                vals = plsc.load_gather(zt_v[j], [sv])
                plsc.addupdate_scatter(acc_v[j], [dv], vals)

        def one_iter(last):
            @plsc.parallel_loop(0, n, step=L, unroll=4)
            def zero_acc(g):
                s = pl.ds(g, L)
                for j in range(fp):
                    acc_v[j][s] = jnp.zeros((L,), jnp.float32)

            edge_pass(scatter_edges)

            @plsc.parallel_loop(0, n, step=L, unroll=2)
            def upd(g):
                s = pl.ds(g, L)
                dv = dinv_v[s]
                for j in range(fp):
                    z = ((1.0 - ALPHA) * dv * (acc_v[j][s] + zt_v[j][s])
                         + ALPHA * h0_v[j][s])
                    zt_v[j][s] = z if last else dv * z

        def k_body(k, c):
            one_iter(False)
            return c

        lax.fori_loop(0, K - 1, k_body, 0)
        one_iter(True)

        for j in range(fp):
            pltpu.sync_copy(zt_v[j], out_hbm.at[f0 + j])

    return prop


def _make_mm(m, kdim, ndim, relu_in, blk_m):
    assert m % blk_m == 0

    def body(x_ref, w_ref, b_ref, o_ref):
        xv = x_ref[...]
        if relu_in:
            xv = jnp.maximum(xv, 0.0)
        o_ref[...] = (
            jnp.dot(xv, w_ref[...], preferred_element_type=jnp.float32)
            + b_ref[...]
        )

    return pl.pallas_call(
        body,
        grid=(m // blk_m,),
        in_specs=[
            pl.BlockSpec((blk_m, kdim), lambda i: (i, 0)),
            pl.BlockSpec((kdim, ndim), lambda i: (0, 0)),
            pl.BlockSpec((1, ndim), lambda i: (0, 0)),
        ],
        out_specs=pl.BlockSpec((blk_m, ndim), lambda i: (i, 0)),
        out_shape=jax.ShapeDtypeStruct((m, ndim), jnp.float32),
    )


def kernel(x, edge_index, W1, b1, W2, b2):
    n, d_in = x.shape
    e = edge_index.shape[1]
    hid = W1.shape[1]
    d_out = W2.shape[1]

    packed = jnp.bitwise_or(
        jnp.left_shift(edge_index[1], jnp.int32(14)), edge_index[0]
    )

    mm1 = _make_mm(n, d_in, hid, relu_in=False, blk_m=1000)
    mm2 = _make_mm(n, hid, d_out, relu_in=True, blk_m=1000)
    prop1 = _make_prop(n, e, hid, n_chunks=10)
    prop2 = _make_prop(n, e, d_out, n_chunks=10)

    h0 = mm1(x, W1, b1.reshape(1, hid))
    z1t = prop1(h0.T, packed)
    h2 = mm2(z1t.T, W2, b2.reshape(1, d_out))
    outt = prop2(h2.T, packed)
    return outt.T


# speedup vs baseline: 22.7847x; 1.0240x over previous
import functools

import jax
import jax.numpy as jnp
from jax import lax
from jax.experimental import pallas as pl
from jax.experimental.pallas import tpu as pltpu
from jax.experimental.pallas import tpu_sc as plsc

ALPHA = 0.1
K = 10
L = 16
NC, NS = 2, 16
NW = NC * NS


def _fast_rsqrt(d):
    i = lax.bitcast_convert_type(d, jnp.int32)
    i = jnp.int32(0x5F3759DF) - lax.shift_right_arithmetic(i, 1)
    y = lax.bitcast_convert_type(i, jnp.float32)
    for _ in range(3):
        y = y * (1.5 - 0.5 * d * y * y)
    return y


def _make_prop(n, e, f, n_chunks, compute_dinv):
    fp = f // NW
    ch = e // n_chunks
    assert fp * NW == f and ch * n_chunks == e and ch % L == 0 and ch % 8 == 0
    assert n <= (1 << 14)
    n_grp = n // L
    c_grp = ch // L
    mesh = plsc.VectorSubcoreMesh(
        core_axis_name="c", subcore_axis_name="s", num_cores=NC, num_subcores=NS
    )

    out_type = jax.ShapeDtypeStruct((f, n), jnp.float32)
    if compute_dinv:
        out_type = (out_type, jax.ShapeDtypeStruct((n,), jnp.float32))

    @functools.partial(
        pl.kernel,
        mesh=mesh,
        compiler_params=pltpu.CompilerParams(needs_layout_passes=False),
        out_type=out_type,
        scratch_types=(
            [pltpu.VMEM((n,), jnp.float32)]
            + [pltpu.VMEM((n,), jnp.float32)] * fp
            + [pltpu.VMEM((n,), jnp.float32)] * fp
            + [pltpu.VMEM((n,), jnp.float32)] * fp
            + [pltpu.VMEM((ch,), jnp.int32)] * 2
            + [
                pltpu.SemaphoreType.DMA,
                pltpu.SemaphoreType.DMA,
            ]
        ),
    )
    def prop(h0t_hbm, edge_hbm, *rest):
        if compute_dinv:
            out_hbm, dinv_hbm = rest[0], rest[1]
            rest = rest[2:]
        else:
            dinv_hbm, out_hbm = rest[0], rest[1]
            rest = rest[2:]
        dinv_v = rest[0]
        h0_v = rest[1:1 + fp]
        zt_v = rest[1 + fp:1 + 2 * fp]
        acc_v = rest[1 + 2 * fp:1 + 3 * fp]
        eb0, eb1, sem0, sem1 = rest[1 + 3 * fp:]
        eb = (eb0, eb1)
        wid = lax.axis_index("s") * NC + lax.axis_index("c")
        f0 = wid * fp
        sems = (sem0, sem1)

        def start_chunk(c, p):
            return pltpu.async_copy(
                edge_hbm.at[pl.ds(c * ch, ch)], eb[p], sems[p])

        def edge_pass(proc):
            pend = start_chunk(0, 0)
            for c in range(n_chunks):
                p = c & 1
                cur = pend
                if c + 1 < n_chunks:
                    pend = start_chunk(c + 1, p ^ 1)
                cur.wait()

                @plsc.parallel_loop(0, ch, step=L, unroll=16)
                def grp_body(g, _p=p):
                    pv = eb[_p][pl.ds(g, L)]
                    sv = lax.bitwise_and(pv, jnp.int32((1 << 14) - 1))
                    dv = lax.shift_right_logical(pv, jnp.int32(14))
                    proc(sv, dv)

        ones = jnp.full((L,), 1.0, jnp.float32)

        if compute_dinv:
            @plsc.parallel_loop(0, n, step=L, unroll=4)
            def init_deg(g):
                acc_v[0][pl.ds(g, L)] = ones

            def hist(sv, dv):
                plsc.addupdate_scatter(acc_v[0], [dv], ones)

            edge_pass(hist)

            @plsc.parallel_loop(0, n, step=L, unroll=2)
            def calc_dinv(g):
                s = pl.ds(g, L)
                dinv_v[s] = _fast_rsqrt(acc_v[0][s])

            @pl.when(wid == 0)
            def _():
                pltpu.sync_copy(dinv_v, dinv_hbm)
        else:
            pltpu.sync_copy(dinv_hbm, dinv_v)

        for j in range(fp):
            pltpu.sync_copy(h0t_hbm.at[f0 + j], h0_v[j])

        @plsc.parallel_loop(0, n, step=L, unroll=2)
        def init_zt(g):
            s = pl.ds(g, L)
            dv = dinv_v[s]
            for j in range(fp):
                zt_v[j][s] = dv * h0_v[j][s]

        zeros = jnp.zeros((L,), jnp.float32)

        @plsc.parallel_loop(0, n, step=L, unroll=4)
        def zero_acc(g):
            s = pl.ds(g, L)
            for j in range(fp):
                acc_v[j][s] = zeros

        def scatter_edges(sv, dv):
            for j in range(fp):
                vals = plsc.load_gather(zt_v[j], [sv])
                plsc.addupdate_scatter(acc_v[j], [dv], vals)

        def one_iter(last):
            edge_pass(scatter_edges)

            @plsc.parallel_loop(0, n, step=L, unroll=2)
            def upd(g):
                s = pl.ds(g, L)
                dv = dinv_v[s]
                for j in range(fp):
                    z = ((1.0 - ALPHA) * dv * (acc_v[j][s] + zt_v[j][s])
                         + ALPHA * h0_v[j][s])
                    zt_v[j][s] = z if last else dv * z
                    if not last:
                        acc_v[j][s] = zeros

        def k_body(k, c):
            one_iter(False)
            return c

        lax.fori_loop(0, K - 1, k_body, 0)
        one_iter(True)

        for j in range(fp):
            pltpu.sync_copy(zt_v[j], out_hbm.at[f0 + j])

    return prop


def _make_mm(m, kdim, ndim, relu_in, blk_m):
    assert m % blk_m == 0

    def body(x_ref, w_ref, b_ref, o_ref):
        xv = x_ref[...]
        if relu_in:
            xv = jnp.maximum(xv, 0.0)
        o_ref[...] = (
            jnp.dot(xv, w_ref[...], preferred_element_type=jnp.float32)
            + b_ref[...]
        )

    return pl.pallas_call(
        body,
        grid=(m // blk_m,),
        in_specs=[
            pl.BlockSpec((blk_m, kdim), lambda i: (i, 0)),
            pl.BlockSpec((kdim, ndim), lambda i: (0, 0)),
            pl.BlockSpec((1, ndim), lambda i: (0, 0)),
        ],
        out_specs=pl.BlockSpec((blk_m, ndim), lambda i: (i, 0)),
        out_shape=jax.ShapeDtypeStruct((m, ndim), jnp.float32),
    )


def kernel(x, edge_index, W1, b1, W2, b2):
    n, d_in = x.shape
    e = edge_index.shape[1]
    hid = W1.shape[1]
    d_out = W2.shape[1]

    packed = jnp.bitwise_or(
        jnp.left_shift(edge_index[1], jnp.int32(14)), edge_index[0]
    )

    mm1 = _make_mm(n, d_in, hid, relu_in=False, blk_m=1000)
    mm2 = _make_mm(n, hid, d_out, relu_in=True, blk_m=1000)
    prop1 = _make_prop(n, e, hid, n_chunks=10, compute_dinv=True)
    prop2 = _make_prop(n, e, d_out, n_chunks=10, compute_dinv=False)

    h0 = mm1(x, W1, b1.reshape(1, hid))
    z1t, dinv = prop1(h0.T, packed)
    h2 = mm2(z1t.T, W2, b2.reshape(1, d_out))
    outt = prop2(h2.T, packed, dinv)
    return outt.T


# speedup vs baseline: 23.8994x; 1.0489x over previous
import functools

import jax
import jax.numpy as jnp
from jax import lax
from jax.experimental import pallas as pl
from jax.experimental.pallas import tpu as pltpu
from jax.experimental.pallas import tpu_sc as plsc

ALPHA = 0.1
K = 10
L = 16
NC, NS = 2, 16
NW = NC * NS


def _fast_rsqrt(d):
    i = lax.bitcast_convert_type(d, jnp.int32)
    i = jnp.int32(0x5F3759DF) - lax.shift_right_arithmetic(i, 1)
    y = lax.bitcast_convert_type(i, jnp.float32)
    for _ in range(3):
        y = y * (1.5 - 0.5 * d * y * y)
    return y


def _make_prop(n, e, f, n_chunks, compute_dinv):
    fp = f // NW
    ch = e // n_chunks
    assert fp * NW == f and ch * n_chunks == e and ch % L == 0 and ch % 8 == 0
    assert n <= (1 << 14)
    n_grp = n // L
    c_grp = ch // L
    mesh = plsc.VectorSubcoreMesh(
        core_axis_name="c", subcore_axis_name="s", num_cores=NC, num_subcores=NS
    )

    out_type = jax.ShapeDtypeStruct((f, n), jnp.float32)
    if compute_dinv:
        out_type = (out_type, jax.ShapeDtypeStruct((n,), jnp.float32))

    @functools.partial(
        pl.kernel,
        mesh=mesh,
        compiler_params=pltpu.CompilerParams(needs_layout_passes=False),
        out_type=out_type,
        scratch_types=(
            [pltpu.VMEM((n,), jnp.float32)]
            + [pltpu.VMEM((n,), jnp.float32)] * fp
            + [pltpu.VMEM((n,), jnp.float32)] * fp
            + [pltpu.VMEM((n,), jnp.float32)] * fp
            + [pltpu.VMEM((ch,), jnp.int32)] * 2
            + [
                pltpu.SemaphoreType.DMA,
                pltpu.SemaphoreType.DMA,
                pltpu.SemaphoreType.DMA,
            ]
        ),
    )
    def prop(h0t_hbm, edge_hbm, *rest):
        if compute_dinv:
            out_hbm, dinv_hbm = rest[0], rest[1]
            rest = rest[2:]
        else:
            dinv_hbm, out_hbm = rest[0], rest[1]
            rest = rest[2:]
        dinv_v = rest[0]
        h0_v = rest[1:1 + fp]
        zt_v = rest[1 + fp:1 + 2 * fp]
        acc_v = rest[1 + 2 * fp:1 + 3 * fp]
        eb0, eb1, sem0, sem1, hsem = rest[1 + 3 * fp:]
        eb = (eb0, eb1)
        wid = lax.axis_index("s") * NC + lax.axis_index("c")
        f0 = wid * fp
        sems = (sem0, sem1)

        def start_chunk(c, p):
            return pltpu.async_copy(
                edge_hbm.at[pl.ds(c * ch, ch)], eb[p], sems[p])

        assert n_chunks % 2 == 0

        def edge_pass(proc, prime_next):
            pend = None
            for c in range(n_chunks):
                p = c & 1
                cur = pend if c else chunk0_pend[0]
                if c + 1 < n_chunks:
                    pend = start_chunk(c + 1, p ^ 1)
                elif prime_next:
                    chunk0_pend[0] = start_chunk(0, 0)
                cur.wait()

                @plsc.parallel_loop(0, ch, step=L, unroll=8)
                def grp_body(g, _p=p):
                    pv = eb[_p][pl.ds(g, L)]
                    sv = lax.bitwise_and(pv, jnp.int32((1 << 14) - 1))
                    dv = lax.shift_right_logical(pv, jnp.int32(14))
                    proc(sv, dv)

        chunk0_pend = [start_chunk(0, 0)]

        in_copies = [
            pltpu.async_copy(h0t_hbm.at[f0 + j], h0_v[j], hsem)
            for j in range(fp)
        ]
        if not compute_dinv:
            in_copies.append(pltpu.async_copy(dinv_hbm, dinv_v, hsem))

        ones = jnp.full((L,), 1.0, jnp.float32)

        if compute_dinv:
            @plsc.parallel_loop(0, n, step=L, unroll=4)
            def init_deg(g):
                acc_v[0][pl.ds(g, L)] = ones

            def hist(sv, dv):
                plsc.addupdate_scatter(acc_v[0], [dv], ones)

            edge_pass(hist, prime_next=True)

            @plsc.parallel_loop(0, n, step=L, unroll=2)
            def calc_dinv(g):
                s = pl.ds(g, L)
                dinv_v[s] = _fast_rsqrt(acc_v[0][s])

            @pl.when(wid == 0)
            def _():
                pltpu.sync_copy(dinv_v, dinv_hbm)

        for cp in in_copies:
            cp.wait()

        @plsc.parallel_loop(0, n, step=L, unroll=2)
        def init_zt(g):
            s = pl.ds(g, L)
            dv = dinv_v[s]
            for j in range(fp):
                zt_v[j][s] = dv * h0_v[j][s]

        zeros = jnp.zeros((L,), jnp.float32)

        @plsc.parallel_loop(0, n, step=L, unroll=4)
        def zero_acc(g):
            s = pl.ds(g, L)
            for j in range(fp):
                acc_v[j][s] = zeros

        def scatter_edges(sv, dv):
            for j in range(fp):
                vals = plsc.load_gather(zt_v[j], [sv])
                plsc.addupdate_scatter(acc_v[j], [dv], vals)

        def one_iter(last):
            edge_pass(scatter_edges, prime_next=not last)

            @plsc.parallel_loop(0, n, step=L, unroll=2)
            def upd(g):
                s = pl.ds(g, L)
                dv = dinv_v[s]
                for j in range(fp):
                    z = ((1.0 - ALPHA) * dv * (acc_v[j][s] + zt_v[j][s])
                         + ALPHA * h0_v[j][s])
                    zt_v[j][s] = z if last else dv * z
                    if not last:
                        acc_v[j][s] = zeros

        def k_body(k, c):
            one_iter(False)
            return c

        lax.fori_loop(0, K - 1, k_body, 0)
        one_iter(True)

        for j in range(fp):
            pltpu.sync_copy(zt_v[j], out_hbm.at[f0 + j])

    return prop


def _make_mm(m, kdim, ndim, relu_in, blk_m):
    assert m % blk_m == 0

    def body(x_ref, w_ref, b_ref, o_ref):
        xv = x_ref[...]
        if relu_in:
            xv = jnp.maximum(xv, 0.0)
        o_ref[...] = (
            jnp.dot(xv, w_ref[...], preferred_element_type=jnp.float32)
            + b_ref[...]
        )

    return pl.pallas_call(
        body,
        grid=(m // blk_m,),
        in_specs=[
            pl.BlockSpec((blk_m, kdim), lambda i: (i, 0)),
            pl.BlockSpec((kdim, ndim), lambda i: (0, 0)),
            pl.BlockSpec((1, ndim), lambda i: (0, 0)),
        ],
        out_specs=pl.BlockSpec((blk_m, ndim), lambda i: (i, 0)),
        out_shape=jax.ShapeDtypeStruct((m, ndim), jnp.float32),
    )


def kernel(x, edge_index, W1, b1, W2, b2):
    n, d_in = x.shape
    e = edge_index.shape[1]
    hid = W1.shape[1]
    d_out = W2.shape[1]

    packed = jnp.bitwise_or(
        jnp.left_shift(edge_index[1], jnp.int32(14)), edge_index[0]
    )

    mm1 = _make_mm(n, d_in, hid, relu_in=False, blk_m=1000)
    mm2 = _make_mm(n, hid, d_out, relu_in=True, blk_m=1000)
    prop1 = _make_prop(n, e, hid, n_chunks=10, compute_dinv=True)
    prop2 = _make_prop(n, e, d_out, n_chunks=10, compute_dinv=False)

    h0 = mm1(x, W1, b1.reshape(1, hid))
    z1t, dinv = prop1(h0.T, packed)
    h2 = mm2(z1t.T, W2, b2.reshape(1, d_out))
    outt = prop2(h2.T, packed, dinv)
    return outt.T
